# baseline (speedup 1.0000x reference)
import functools

import jax
import jax.numpy as jnp
from jax import lax
from jax.experimental import pallas as pl
from jax.experimental.pallas import tpu as pltpu
from jax.experimental.pallas import tpu_sc as plsc

N = 16384
F = 128
H = 50
G = 100000

NC = 2
NS = 16
L = 16
NW = NC * NS
BPW = N // NW
BLK = 1024

_SELU_ALPHA = 1.6732632423543772848170429916717
_SELU_SCALE = 1.0507009873554804934193349852946


def _tc_body(x_ref, w_ref, b_ref, g_ref, act_ref, idx_ref):
    pre = lax.dot_general(w_ref[...], x_ref[...], (((0,), (1,)), ((), ())),
                          preferred_element_type=jnp.float32)
    pre = pre + b_ref[...]
    act_ref[...] = _SELU_SCALE * jnp.where(
        pre > 0, pre, _SELU_ALPHA * (jnp.exp(pre) - 1.0))
    g3 = g_ref[0, 0, :].reshape(BLK // BPW, 1, BPW)
    hoff = lax.broadcasted_iota(jnp.int32, (1, H, 1), 1) * G
    idx_ref[...] = g3 + hoff


def _tc_mlp_idx(x, w1, b1_2d, gene3):
    return pl.pallas_call(
        _tc_body,
        grid=(N // BLK,),
        in_specs=[
            pl.BlockSpec((BLK, F), lambda i: (i, 0)),
            pl.BlockSpec((F, H), lambda i: (0, 0)),
            pl.BlockSpec((H, 1), lambda i: (0, 0)),
            pl.BlockSpec((1, 1, BLK), lambda i: (i, 0, 0)),
        ],
        out_specs=[
            pl.BlockSpec((H, BLK), lambda i: (0, i)),
            pl.BlockSpec((BLK // BPW, H, BPW), lambda i: (i, 0, 0)),
        ],
        out_shape=[
            jax.ShapeDtypeStruct((H, N), jnp.float32),
            jax.ShapeDtypeStruct((NW, H, BPW), jnp.int32),
        ],
    )(x, w1, b1_2d, gene3)


_sc_mesh = plsc.VectorSubcoreMesh(
    core_axis_name="c", subcore_axis_name="s", num_cores=NC, num_subcores=NS)


@functools.partial(
    pl.kernel,
    out_type=(
        jax.ShapeDtypeStruct((N,), jnp.float32),
        jax.ShapeDtypeStruct((N,), jnp.float32),
        jax.ShapeDtypeStruct((N,), jnp.float32),
    ),
    mesh=_sc_mesh,
    scratch_types=[
        pltpu.VMEM((H * BPW,), jnp.int32),
        pltpu.VMEM((H * BPW,), jnp.float32),
        pltpu.VMEM((H * BPW,), jnp.float32),
        pltpu.VMEM((BPW,), jnp.float32),
        pltpu.VMEM((BPW,), jnp.float32),
        pltpu.VMEM((BPW,), jnp.float32),
        pltpu.VMEM((BPW,), jnp.float32),
        pltpu.SemaphoreType.DMA,
        pltpu.SemaphoreType.DMA,
        pltpu.SemaphoreType.DMA,
    ],
)
def _sc_gather_dot(idx3_hbm, act_hbm, table_hbm, bias_hbm, phi_hbm, pz_hbm,
                   mu_out, phi_out, pz_out,
                   idx_v, a_v, w_v, bias_v, phi_v, pz_v, mu_v,
                   sem_w, sem_s, sem_a):
    wid = lax.axis_index("s") * NC + lax.axis_index("c")
    base = wid * BPW

    pltpu.sync_copy(idx3_hbm.at[pl.ds(wid * H * BPW, H * BPW)], idx_v)

    copies = []
    for h in range(H):
        copies.append(pltpu.async_copy(
            table_hbm.at[idx_v.at[pl.ds(h * BPW, BPW)]],
            w_v.at[pl.ds(h * BPW, BPW)], sem_w))
        copies.append(pltpu.async_copy(
            act_hbm.at[pl.ds(h * N + base, BPW)],
            a_v.at[pl.ds(h * BPW, BPW)], sem_a))
    gene_idx = idx_v.at[pl.ds(0, BPW)]
    copies.append(pltpu.async_copy(bias_hbm.at[gene_idx], bias_v, sem_s))
    copies.append(pltpu.async_copy(phi_hbm.at[gene_idx], phi_v, sem_s))
    copies.append(pltpu.async_copy(pz_hbm.at[gene_idx], pz_v, sem_s))
    for c in copies:
        c.wait()

    for j in range(BPW // L):
        off = j * L

        def body(h, acc):
            a = a_v[pl.ds(h * BPW + off, L)]
            w = w_v[pl.ds(h * BPW + off, L)]
            return acc + a * w

        acc = lax.fori_loop(0, H, body, jnp.zeros((L,), jnp.float32))
        mu_v[pl.ds(off, L)] = acc + bias_v[pl.ds(off, L)]

    pltpu.sync_copy(mu_v, mu_out.at[pl.ds(base, BPW)])
    pltpu.sync_copy(phi_v, phi_out.at[pl.ds(base, BPW)])
    pltpu.sync_copy(pz_v, pz_out.at[pl.ds(base, BPW)])


def kernel(gene_index_tensor_n, cell_index_tensor_n, cell_features_nf,
           total_obs_reads_per_cell_tensor_n, downsampling_rate_tensor_n,
           W1, b1, readout_weight_hg, readout_bias_g,
           log_phi_e_hi_g, logit_p_zero_e_hi_g):
    del cell_index_tensor_n, total_obs_reads_per_cell_tensor_n
    del downsampling_rate_tensor_n
    gene3 = gene_index_tensor_n.astype(jnp.int32).reshape(N // BLK, 1, BLK)
    act_t, idx3 = _tc_mlp_idx(cell_features_nf, W1, b1.reshape(H, 1), gene3)
    mu, phi, pz = _sc_gather_dot(
        idx3.reshape(NW * H * BPW), act_t.reshape(H * N),
        readout_weight_hg.reshape(H * G), readout_bias_g,
        log_phi_e_hi_g, logit_p_zero_e_hi_g)
    return mu, phi, pz
